# Initial kernel scaffold; baseline (speedup 1.0000x reference)
import jax
import jax.numpy as jnp
from jax.experimental import pallas as pl


def kernel(w, a, s, d, space, shift, mouse_1, mouse_2, dx, dy, w_tab, a_tab, s_tab, d_tab, space_tab, shift_tab, m1_tab, m2_tab, dx_W1, dx_b1, dx_W2, dx_b2, dy_W1, dy_b1, dy_W2, dy_b2, ffn_W, ffn_b, ln_w):
    raise NotImplementedError("write your pallas kernel here")


# speedup vs baseline: 3.8380x; 3.8380x over previous
import functools
import math

import jax
import jax.numpy as jnp
from jax.experimental import pallas as pl
from jax.experimental.pallas import tpu as pltpu

HID = 64
GROUP = 4
OUT = 256
NSRC = 10
_BIT_SRC = [0, 1, 2, 3, 4, 5, None, None, 6, 7]


def _prologue_body(tabs_ref, dxW2_ref, dyW2_ref, dxb2_ref, dyb2_ref,
                   ffnW_ref, ffnb_ref, C_ref, DJ_ref, MX_ref, MY_ref):
    W = ffnW_ref[...]
    T0 = tabs_ref[:, 0, :]
    DT = tabs_ref[:, 1, :] - T0
    T0t = jnp.concatenate([T0] * GROUP, axis=1)
    b2x = jnp.concatenate([dxb2_ref[...]] * GROUP, axis=1)
    b2y = jnp.concatenate([dyb2_ref[...]] * GROUP, axis=1)
    SRC = jnp.concatenate([T0t[0:6], b2x, b2y, T0t[6:8]], axis=0)
    BASE10 = jnp.dot(SRC, W, preferred_element_type=jnp.float32)

    NROW = C_ref.shape[0]
    row = jax.lax.broadcasted_iota(jnp.int32, (NROW, OUT), 0).astype(jnp.float32)
    col = jax.lax.broadcasted_iota(jnp.int32, (NROW, OUT), 1)
    j = col // HID
    d = col % HID
    p = row * float(GROUP) + j.astype(jnp.float32)
    dd = ((d // 2) * 2).astype(jnp.float32)
    freq = jnp.exp(dd * (-math.log(10000.0) / HID))
    ang = p * freq
    pe_r = jnp.where(d % 2 == 0, jnp.sin(ang), jnp.cos(ang))

    C = jnp.dot(pe_r, W, preferred_element_type=jnp.float32) + ffnb_ref[...]
    C = C + jnp.broadcast_to(BASE10[:, None, :], (NSRC, NROW // NSRC, OUT)
                             ).reshape(NROW, OUT)
    C_ref[...] = C

    for jj in range(GROUP):
        Wj = W[HID * jj:HID * (jj + 1), :]
        DJ_ref[jj] = jnp.dot(DT, Wj, preferred_element_type=jnp.float32)
        MX_ref[jj] = jnp.dot(dxW2_ref[...], Wj, preferred_element_type=jnp.float32)
        MY_ref[jj] = jnp.dot(dyW2_ref[...], Wj, preferred_element_type=jnp.float32)


def _main_body(BITS_ref, DX4_ref, DY4_ref, dxW1_ref, dxb1_ref, dyW1_ref,
               dyb1_ref, C_ref, DJ_ref, MX_ref, MY_ref, lnw_ref, out_ref):
    Bt = out_ref.shape[0]
    NG = out_ref.shape[1] // NSRC
    lnw = lnw_ref[...].reshape(1, 1, OUT)

    for s10 in range(NSRC):
        cpart = C_ref[NG * s10:NG * (s10 + 1), :][None]
        acc = jnp.broadcast_to(cpart, (Bt, NG, OUT))
        if s10 in (6, 7):
            R = DX4_ref if s10 == 6 else DY4_ref
            W1 = (dxW1_ref if s10 == 6 else dyW1_ref)[...].reshape(1, 1, HID)
            b1 = (dxb1_ref if s10 == 6 else dyb1_ref)[...].reshape(1, 1, HID)
            M = MX_ref if s10 == 6 else MY_ref
            for jj in range(GROUP):
                v = R[:, jj]
                h = jnp.maximum(v * W1 + b1, 0.0)
                acc = acc + jnp.dot(
                    h.reshape(Bt * NG, HID), M[jj],
                    preferred_element_type=jnp.float32).reshape(Bt, NG, OUT)
        else:
            m = _BIT_SRC[s10]
            for jj in range(GROUP):
                bj = BITS_ref[:, m, jj]
                acc = acc + bj * DJ_ref[jj, m].reshape(1, 1, OUT)
        ms = jnp.mean(acc * acc, axis=-1, keepdims=True)
        out_ref[:, NG * s10:NG * (s10 + 1), :] = (
            acc * jax.lax.rsqrt(ms + 1e-6) * lnw)


def kernel(w, a, s, d, space, shift, mouse_1, mouse_2, dx, dy, w_tab, a_tab,
           s_tab, d_tab, space_tab, shift_tab, m1_tab, m2_tab, dx_W1, dx_b1,
           dx_W2, dx_b2, dy_W1, dy_b1, dy_W2, dy_b2, ffn_W, ffn_b, ln_w):
    B, T = w.shape
    NG = T // GROUP
    NROW = NSRC * NG
    f32 = jnp.float32

    tabs = jnp.stack([w_tab, a_tab, s_tab, d_tab, space_tab, shift_tab,
                      m1_tab, m2_tab])
    bits = jnp.stack([w, a, s, d, space, shift, mouse_1, mouse_2], axis=1)
    BITS = bits.reshape(B, 8, NG, GROUP).transpose(0, 1, 3, 2)[..., None]
    BITS = BITS.astype(f32)
    DX4 = dx.reshape(B, NG, GROUP).transpose(0, 2, 1)[..., None]
    DY4 = dy.reshape(B, NG, GROUP).transpose(0, 2, 1)[..., None]

    C, DJ, MX, MY = pl.pallas_call(
        _prologue_body,
        out_shape=[
            jax.ShapeDtypeStruct((NROW, OUT), f32),
            jax.ShapeDtypeStruct((GROUP, 8, OUT), f32),
            jax.ShapeDtypeStruct((GROUP, HID, OUT), f32),
            jax.ShapeDtypeStruct((GROUP, HID, OUT), f32),
        ],
    )(tabs, dx_W2, dy_W2, dx_b2.reshape(1, HID), dy_b2.reshape(1, HID),
      ffn_W, ffn_b.reshape(1, OUT))

    Bt = 16
    grid = (B // Bt,)
    full = lambda shape: pl.BlockSpec(shape, lambda i: (0,) * len(shape))
    out = pl.pallas_call(
        _main_body,
        grid=grid,
        in_specs=[
            pl.BlockSpec((Bt, 8, GROUP, NG, 1), lambda i: (i, 0, 0, 0, 0)),
            pl.BlockSpec((Bt, GROUP, NG, 1), lambda i: (i, 0, 0, 0)),
            pl.BlockSpec((Bt, GROUP, NG, 1), lambda i: (i, 0, 0, 0)),
            full((1, HID)), full((1, HID)), full((1, HID)), full((1, HID)),
            full((NROW, OUT)), full((GROUP, 8, OUT)),
            full((GROUP, HID, OUT)), full((GROUP, HID, OUT)),
            full((1, OUT)),
        ],
        out_specs=pl.BlockSpec((Bt, NROW, OUT), lambda i: (i, 0, 0)),
        out_shape=jax.ShapeDtypeStruct((B, NROW, OUT), f32),
    )(BITS, DX4, DY4, dx_W1, dx_b1.reshape(1, HID), dy_W1,
      dy_b1.reshape(1, HID), C, DJ, MX, MY, ln_w.reshape(1, OUT))
    return out


# speedup vs baseline: 6.4877x; 1.6904x over previous
import math

import jax
import jax.numpy as jnp
from jax.experimental import pallas as pl

HID = 64
GROUP = 4
OUT = 256
NSRC = 10
_BIT_SRC = [0, 1, 2, 3, 4, 5, None, None, 6, 7]


def _prologue_body(tabs_ref, dxW2_ref, dyW2_ref, dxb2_ref, dyb2_ref,
                   ffnW_ref, ffnb_ref, C_ref, DJ_ref, MX_ref, MY_ref):
    W = ffnW_ref[...]
    T0 = tabs_ref[:, 0, :]
    DT = tabs_ref[:, 1, :] - T0
    T0t = jnp.concatenate([T0] * GROUP, axis=1)
    b2x = jnp.concatenate([dxb2_ref[...]] * GROUP, axis=1)
    b2y = jnp.concatenate([dyb2_ref[...]] * GROUP, axis=1)
    SRC = jnp.concatenate([T0t[0:6], b2x, b2y, T0t[6:8]], axis=0)
    BASE10 = jnp.dot(SRC, W, preferred_element_type=jnp.float32)

    NROW = C_ref.shape[0]
    row = jax.lax.broadcasted_iota(jnp.int32, (NROW, OUT), 0).astype(jnp.float32)
    col = jax.lax.broadcasted_iota(jnp.int32, (NROW, OUT), 1)
    j = col // HID
    d = col % HID
    p = row * float(GROUP) + j.astype(jnp.float32)
    dd = ((d // 2) * 2).astype(jnp.float32)
    freq = jnp.exp(dd * (-math.log(10000.0) / HID))
    ang = p * freq
    pe_r = jnp.where(d % 2 == 0, jnp.sin(ang), jnp.cos(ang))

    C = jnp.dot(pe_r, W, preferred_element_type=jnp.float32) + ffnb_ref[...]
    C = C + jnp.broadcast_to(BASE10[:, None, :], (NSRC, NROW // NSRC, OUT)
                             ).reshape(NROW, OUT)
    C_ref[...] = C

    for jj in range(GROUP):
        Wj = W[HID * jj:HID * (jj + 1), :]
        DJ_ref[:, jj, :] = jnp.dot(DT, Wj, preferred_element_type=jnp.float32)
        MX_ref[jj] = jnp.dot(dxW2_ref[...], Wj, preferred_element_type=jnp.float32)
        MY_ref[jj] = jnp.dot(dyW2_ref[...], Wj, preferred_element_type=jnp.float32)


def _main_body(X_ref, dxW1_ref, dxb1_ref, dyW1_ref, dyb1_ref, C_ref, DJ_ref,
               MX_ref, MY_ref, lnw_ref, out_ref):
    Bt = out_ref.shape[0]
    NG = out_ref.shape[1] // NSRC
    lnw = lnw_ref[...].reshape(1, 1, OUT)

    for s10 in range(NSRC):
        Xs = X_ref[:, NG * s10:NG * (s10 + 1), :]
        cpart = C_ref[NG * s10:NG * (s10 + 1), :][None]
        if s10 in (6, 7):
            W1 = (dxW1_ref if s10 == 6 else dyW1_ref)[...].reshape(1, 1, HID)
            b1 = (dxb1_ref if s10 == 6 else dyb1_ref)[...].reshape(1, 1, HID)
            M = MX_ref if s10 == 6 else MY_ref
            acc = jnp.broadcast_to(cpart, (Bt, NG, OUT))
            for jj in range(GROUP):
                v = Xs[:, :, jj:jj + 1]
                h = jnp.maximum(v * W1 + b1, 0.0)
                acc = acc + jnp.dot(
                    h.reshape(Bt * NG, HID), M[jj],
                    preferred_element_type=jnp.float32).reshape(Bt, NG, OUT)
        else:
            m = _BIT_SRC[s10]
            acc = jnp.dot(
                Xs.reshape(Bt * NG, GROUP), DJ_ref[m],
                preferred_element_type=jnp.float32).reshape(Bt, NG, OUT)
            acc = acc + cpart
        ms = jnp.mean(acc * acc, axis=-1, keepdims=True)
        out_ref[:, NG * s10:NG * (s10 + 1), :] = (
            acc * jax.lax.rsqrt(ms + 1e-6) * lnw)


def kernel(w, a, s, d, space, shift, mouse_1, mouse_2, dx, dy, w_tab, a_tab,
           s_tab, d_tab, space_tab, shift_tab, m1_tab, m2_tab, dx_W1, dx_b1,
           dx_W2, dx_b2, dy_W1, dy_b1, dy_W2, dy_b2, ffn_W, ffn_b, ln_w):
    B, T = w.shape
    NG = T // GROUP
    NROW = NSRC * NG
    f32 = jnp.float32

    tabs = jnp.stack([w_tab, a_tab, s_tab, d_tab, space_tab, shift_tab,
                      m1_tab, m2_tab])
    X = jnp.concatenate(
        [w.astype(f32), a.astype(f32), s.astype(f32), d.astype(f32),
         space.astype(f32), shift.astype(f32), dx, dy,
         mouse_1.astype(f32), mouse_2.astype(f32)], axis=1)
    X = X.reshape(B, NROW, GROUP)

    C, DJ, MX, MY = pl.pallas_call(
        _prologue_body,
        out_shape=[
            jax.ShapeDtypeStruct((NROW, OUT), f32),
            jax.ShapeDtypeStruct((8, GROUP, OUT), f32),
            jax.ShapeDtypeStruct((GROUP, HID, OUT), f32),
            jax.ShapeDtypeStruct((GROUP, HID, OUT), f32),
        ],
    )(tabs, dx_W2, dy_W2, dx_b2.reshape(1, HID), dy_b2.reshape(1, HID),
      ffn_W, ffn_b.reshape(1, OUT))

    Bt = 16
    grid = (B // Bt,)
    full = lambda shape: pl.BlockSpec(shape, lambda i: (0,) * len(shape))
    out = pl.pallas_call(
        _main_body,
        grid=grid,
        in_specs=[
            pl.BlockSpec((Bt, NROW, GROUP), lambda i: (i, 0, 0)),
            full((1, HID)), full((1, HID)), full((1, HID)), full((1, HID)),
            full((NROW, OUT)), full((8, GROUP, OUT)),
            full((GROUP, HID, OUT)), full((GROUP, HID, OUT)),
            full((1, OUT)),
        ],
        out_specs=pl.BlockSpec((Bt, NROW, OUT), lambda i: (i, 0, 0)),
        out_shape=jax.ShapeDtypeStruct((B, NROW, OUT), f32),
    )(X, dx_W1, dx_b1.reshape(1, HID), dy_W1, dy_b1.reshape(1, HID),
      C, DJ, MX, MY, ln_w.reshape(1, OUT))
    return out


# speedup vs baseline: 7.2978x; 1.1249x over previous
import math

import jax
import jax.numpy as jnp
from jax.experimental import pallas as pl

HID = 64
GROUP = 4
OUT = 256
NSRC = 10
_BIT_SRC = [0, 1, 2, 3, 4, 5, None, None, 6, 7]


def _prologue_body(tabs_ref, dxW1_ref, dyW1_ref, dxb1_ref, dyb1_ref,
                   dxW2_ref, dyW2_ref, dxb2_ref, dyb2_ref,
                   ffnW_ref, ffnb_ref,
                   C_ref, DJ_ref, KX_ref, KY_ref, B1_ref, MX_ref, MY_ref):
    W = ffnW_ref[...]
    T0 = tabs_ref[:, 0, :]
    DT = tabs_ref[:, 1, :] - T0
    T0t = jnp.concatenate([T0] * GROUP, axis=1)
    b2x = jnp.concatenate([dxb2_ref[...]] * GROUP, axis=1)
    b2y = jnp.concatenate([dyb2_ref[...]] * GROUP, axis=1)
    SRC = jnp.concatenate([T0t[0:6], b2x, b2y, T0t[6:8]], axis=0)
    BASE10 = jnp.dot(SRC, W, preferred_element_type=jnp.float32)

    NROW = C_ref.shape[0] * C_ref.shape[1]
    row = jax.lax.broadcasted_iota(jnp.int32, (NROW, OUT), 0).astype(jnp.float32)
    col = jax.lax.broadcasted_iota(jnp.int32, (NROW, OUT), 1)
    j = col // HID
    d = col % HID
    p = row * float(GROUP) + j.astype(jnp.float32)
    dd = ((d // 2) * 2).astype(jnp.float32)
    freq = jnp.exp(dd * (-math.log(10000.0) / HID))
    ang = p * freq
    pe_r = jnp.where(d % 2 == 0, jnp.sin(ang), jnp.cos(ang))

    C = jnp.dot(pe_r, W, preferred_element_type=jnp.float32) + ffnb_ref[...]
    C = C + jnp.broadcast_to(BASE10[:, None, :], (NSRC, NROW // NSRC, OUT)
                             ).reshape(NROW, OUT)
    C_ref[...] = C.reshape(C_ref.shape)

    zero = jnp.zeros((1, HID), jnp.float32)
    rowsx, rowsy = [], []
    for jj in range(GROUP):
        px = [dxW1_ref[...] if k == jj else zero for k in range(GROUP)]
        py = [dyW1_ref[...] if k == jj else zero for k in range(GROUP)]
        rowsx.append(jnp.concatenate(px, axis=1))
        rowsy.append(jnp.concatenate(py, axis=1))
    KX_ref[...] = jnp.concatenate(rowsx, axis=0)
    KY_ref[...] = jnp.concatenate(rowsy, axis=0)
    B1_ref[...] = jnp.concatenate(
        [jnp.concatenate([dxb1_ref[...]] * GROUP, axis=1),
         jnp.concatenate([dyb1_ref[...]] * GROUP, axis=1)], axis=0)

    mx, my = [], []
    for jj in range(GROUP):
        Wj = W[HID * jj:HID * (jj + 1), :]
        DJ_ref[:, jj, :] = jnp.dot(DT, Wj, preferred_element_type=jnp.float32)
        mx.append(jnp.dot(dxW2_ref[...], Wj, preferred_element_type=jnp.float32))
        my.append(jnp.dot(dyW2_ref[...], Wj, preferred_element_type=jnp.float32))
    MX_ref[...] = jnp.concatenate(mx, axis=0)
    MY_ref[...] = jnp.concatenate(my, axis=0)


def _main_body(X_ref, C_ref, DJ_ref, KX_ref, KY_ref, B1_ref, MX_ref, MY_ref,
               lnw_ref, out_ref):
    Bt = out_ref.shape[0]
    NG = out_ref.shape[1] // NSRC
    lnw = lnw_ref[...].reshape(1, 1, OUT)

    for s10 in range(NSRC):
        Xs = X_ref[:, NG * s10:NG * (s10 + 1), :].reshape(Bt * NG, GROUP)
        cpart = C_ref[s10][None]
        if s10 in (6, 7):
            K1 = KX_ref if s10 == 6 else KY_ref
            M = MX_ref if s10 == 6 else MY_ref
            b1 = B1_ref[s10 - 6][None]
            pre = jnp.dot(Xs, K1[...], preferred_element_type=jnp.float32)
            h = jnp.maximum(pre + b1, 0.0)
            acc = jnp.dot(h, M[...], preferred_element_type=jnp.float32)
        else:
            acc = jnp.dot(Xs, DJ_ref[_BIT_SRC[s10]],
                          preferred_element_type=jnp.float32)
        acc = acc.reshape(Bt, NG, OUT) + cpart
        ms = jnp.mean(acc * acc, axis=-1, keepdims=True)
        out_ref[:, NG * s10:NG * (s10 + 1), :] = (
            acc * jax.lax.rsqrt(ms + 1e-6) * lnw)


def kernel(w, a, s, d, space, shift, mouse_1, mouse_2, dx, dy, w_tab, a_tab,
           s_tab, d_tab, space_tab, shift_tab, m1_tab, m2_tab, dx_W1, dx_b1,
           dx_W2, dx_b2, dy_W1, dy_b1, dy_W2, dy_b2, ffn_W, ffn_b, ln_w):
    B, T = w.shape
    NG = T // GROUP
    NROW = NSRC * NG
    f32 = jnp.float32

    tabs = jnp.stack([w_tab, a_tab, s_tab, d_tab, space_tab, shift_tab,
                      m1_tab, m2_tab])
    X = jnp.concatenate(
        [w.astype(f32), a.astype(f32), s.astype(f32), d.astype(f32),
         space.astype(f32), shift.astype(f32), dx, dy,
         mouse_1.astype(f32), mouse_2.astype(f32)], axis=1)
    X = X.reshape(B, NROW, GROUP)

    C, DJ, KX, KY, B1, MX, MY = pl.pallas_call(
        _prologue_body,
        out_shape=[
            jax.ShapeDtypeStruct((NSRC, NG, OUT), f32),
            jax.ShapeDtypeStruct((8, GROUP, OUT), f32),
            jax.ShapeDtypeStruct((GROUP, OUT), f32),
            jax.ShapeDtypeStruct((GROUP, OUT), f32),
            jax.ShapeDtypeStruct((2, OUT), f32),
            jax.ShapeDtypeStruct((OUT, OUT), f32),
            jax.ShapeDtypeStruct((OUT, OUT), f32),
        ],
    )(tabs, dx_W1, dy_W1, dx_b1.reshape(1, HID), dy_b1.reshape(1, HID),
      dx_W2, dy_W2, dx_b2.reshape(1, HID), dy_b2.reshape(1, HID),
      ffn_W, ffn_b.reshape(1, OUT))

    Bt = 16
    grid = (B // Bt,)
    full = lambda shape: pl.BlockSpec(shape, lambda i: (0,) * len(shape))
    out = pl.pallas_call(
        _main_body,
        grid=grid,
        in_specs=[
            pl.BlockSpec((Bt, NROW, GROUP), lambda i: (i, 0, 0)),
            full((NSRC, NG, OUT)), full((8, GROUP, OUT)),
            full((GROUP, OUT)), full((GROUP, OUT)), full((2, OUT)),
            full((OUT, OUT)), full((OUT, OUT)),
            full((1, OUT)),
        ],
        out_specs=pl.BlockSpec((Bt, NROW, OUT), lambda i: (i, 0, 0)),
        out_shape=jax.ShapeDtypeStruct((B, NROW, OUT), f32),
    )(X, C, DJ, KX, KY, B1, MX, MY, ln_w.reshape(1, OUT))
    return out


# speedup vs baseline: 7.3232x; 1.0035x over previous
import math

import jax
import jax.numpy as jnp
from jax.experimental import pallas as pl

HID = 64
GROUP = 4
OUT = 256
NSRC = 10
_BIT_SRC = [0, 1, 2, 3, 4, 5, None, None, 6, 7]


def _prologue_body(tabs_ref, dxW1_ref, dyW1_ref, dxb1_ref, dyb1_ref,
                   dxW2_ref, dyW2_ref, dxb2_ref, dyb2_ref,
                   ffnW_ref, ffnb_ref,
                   C_ref, DJ_ref, KX_ref, KY_ref, B1_ref, MX_ref, MY_ref):
    W = ffnW_ref[...]
    T0 = tabs_ref[:, 0, :]
    DT = tabs_ref[:, 1, :] - T0
    T0t = jnp.concatenate([T0] * GROUP, axis=1)
    b2x = jnp.concatenate([dxb2_ref[...]] * GROUP, axis=1)
    b2y = jnp.concatenate([dyb2_ref[...]] * GROUP, axis=1)
    SRC = jnp.concatenate([T0t[0:6], b2x, b2y, T0t[6:8]], axis=0)
    BASE10 = jnp.dot(SRC, W, preferred_element_type=jnp.float32)

    NROW = C_ref.shape[0] * C_ref.shape[1]
    row = jax.lax.broadcasted_iota(jnp.int32, (NROW, OUT), 0).astype(jnp.float32)
    col = jax.lax.broadcasted_iota(jnp.int32, (NROW, OUT), 1)
    j = col // HID
    d = col % HID
    p = row * float(GROUP) + j.astype(jnp.float32)
    dd = ((d // 2) * 2).astype(jnp.float32)
    freq = jnp.exp(dd * (-math.log(10000.0) / HID))
    ang = p * freq
    pe_r = jnp.where(d % 2 == 0, jnp.sin(ang), jnp.cos(ang))

    C = jnp.dot(pe_r, W, preferred_element_type=jnp.float32) + ffnb_ref[...]
    C = C + jnp.broadcast_to(BASE10[:, None, :], (NSRC, NROW // NSRC, OUT)
                             ).reshape(NROW, OUT)
    C_ref[...] = C.reshape(C_ref.shape)

    zero = jnp.zeros((1, HID), jnp.float32)
    rowsx, rowsy = [], []
    for jj in range(GROUP):
        px = [dxW1_ref[...] if k == jj else zero for k in range(GROUP)]
        py = [dyW1_ref[...] if k == jj else zero for k in range(GROUP)]
        rowsx.append(jnp.concatenate(px, axis=1))
        rowsy.append(jnp.concatenate(py, axis=1))
    KX_ref[...] = jnp.concatenate(rowsx, axis=0)
    KY_ref[...] = jnp.concatenate(rowsy, axis=0)
    B1_ref[...] = jnp.concatenate(
        [jnp.concatenate([dxb1_ref[...]] * GROUP, axis=1),
         jnp.concatenate([dyb1_ref[...]] * GROUP, axis=1)], axis=0)

    mx, my = [], []
    for jj in range(GROUP):
        Wj = W[HID * jj:HID * (jj + 1), :]
        DJ_ref[:, jj, :] = jnp.dot(DT, Wj, preferred_element_type=jnp.float32)
        mx.append(jnp.dot(dxW2_ref[...], Wj, preferred_element_type=jnp.float32))
        my.append(jnp.dot(dyW2_ref[...], Wj, preferred_element_type=jnp.float32))
    MX_ref[...] = jnp.concatenate(mx, axis=0)
    MY_ref[...] = jnp.concatenate(my, axis=0)


def _main_body(X_ref, C_ref, DJ_ref, KX_ref, KY_ref, B1_ref, MX_ref, MY_ref,
               lnw_ref, out_ref):
    Bt = out_ref.shape[0]
    NG = out_ref.shape[1] // NSRC
    lnw = lnw_ref[...].reshape(1, 1, OUT)

    for s10 in range(NSRC):
        Xs = X_ref[:, NG * s10:NG * (s10 + 1), :].reshape(Bt * NG, GROUP)
        cpart = C_ref[s10][None]
        if s10 in (6, 7):
            K1 = KX_ref if s10 == 6 else KY_ref
            M = MX_ref if s10 == 6 else MY_ref
            b1 = B1_ref[s10 - 6][None]
            pre = jnp.dot(Xs, K1[...], preferred_element_type=jnp.float32)
            h = jnp.maximum(pre + b1, 0.0)
            acc = jnp.dot(h, M[...], preferred_element_type=jnp.float32)
        else:
            acc = jnp.dot(Xs, DJ_ref[_BIT_SRC[s10]],
                          preferred_element_type=jnp.float32)
        acc = acc.reshape(Bt, NG, OUT) + cpart
        ms = jnp.mean(acc * acc, axis=-1, keepdims=True)
        out_ref[:, NG * s10:NG * (s10 + 1), :] = (
            acc * jax.lax.rsqrt(ms + 1e-6) * lnw)


def kernel(w, a, s, d, space, shift, mouse_1, mouse_2, dx, dy, w_tab, a_tab,
           s_tab, d_tab, space_tab, shift_tab, m1_tab, m2_tab, dx_W1, dx_b1,
           dx_W2, dx_b2, dy_W1, dy_b1, dy_W2, dy_b2, ffn_W, ffn_b, ln_w):
    B, T = w.shape
    NG = T // GROUP
    NROW = NSRC * NG
    f32 = jnp.float32

    tabs = jnp.stack([w_tab, a_tab, s_tab, d_tab, space_tab, shift_tab,
                      m1_tab, m2_tab])
    X = jnp.concatenate(
        [w.astype(f32), a.astype(f32), s.astype(f32), d.astype(f32),
         space.astype(f32), shift.astype(f32), dx, dy,
         mouse_1.astype(f32), mouse_2.astype(f32)], axis=1)
    X = X.reshape(B, NROW, GROUP)

    C, DJ, KX, KY, B1, MX, MY = pl.pallas_call(
        _prologue_body,
        out_shape=[
            jax.ShapeDtypeStruct((NSRC, NG, OUT), f32),
            jax.ShapeDtypeStruct((8, GROUP, OUT), f32),
            jax.ShapeDtypeStruct((GROUP, OUT), f32),
            jax.ShapeDtypeStruct((GROUP, OUT), f32),
            jax.ShapeDtypeStruct((2, OUT), f32),
            jax.ShapeDtypeStruct((OUT, OUT), f32),
            jax.ShapeDtypeStruct((OUT, OUT), f32),
        ],
    )(tabs, dx_W1, dy_W1, dx_b1.reshape(1, HID), dy_b1.reshape(1, HID),
      dx_W2, dy_W2, dx_b2.reshape(1, HID), dy_b2.reshape(1, HID),
      ffn_W, ffn_b.reshape(1, OUT))

    Bt = 32
    grid = (B // Bt,)
    full = lambda shape: pl.BlockSpec(shape, lambda i: (0,) * len(shape))
    out = pl.pallas_call(
        _main_body,
        grid=grid,
        in_specs=[
            pl.BlockSpec((Bt, NROW, GROUP), lambda i: (i, 0, 0)),
            full((NSRC, NG, OUT)), full((8, GROUP, OUT)),
            full((GROUP, OUT)), full((GROUP, OUT)), full((2, OUT)),
            full((OUT, OUT)), full((OUT, OUT)),
            full((1, OUT)),
        ],
        out_specs=pl.BlockSpec((Bt, NROW, OUT), lambda i: (i, 0, 0)),
        out_shape=jax.ShapeDtypeStruct((B, NROW, OUT), f32),
    )(X, C, DJ, KX, KY, B1, MX, MY, ln_w.reshape(1, OUT))
    return out
